# Initial kernel scaffold; baseline (speedup 1.0000x reference)
import jax
import jax.numpy as jnp
from jax.experimental import pallas as pl


def kernel(x, edge_index, batch, params):
    raise NotImplementedError("write your pallas kernel here")


# speedup vs baseline: 22.7635x; 22.7635x over previous
import functools

import jax
import jax.numpy as jnp
from jax import lax
from jax.experimental import pallas as pl
from jax.experimental.pallas import tpu as pltpu
from jax.experimental.pallas import tpu_sc as plsc

N_NODES = 100000
N_PAD = 100352
H = 16
G = 64
LANES = 16
KC = 2048
NG = KC // 128
KCB = 1024
NGB = KCB // 128
NW = 32
RPS = N_PAD // 16
NEG = -1e30
BLK = 2048
NB = N_PAD // BLK
F32 = jnp.float32


def _sc_alpha(src2d, dst2d, xl, xr, att, e2, cpw):
    e_pad = src2d.shape[0] * 128
    mesh = plsc.VectorSubcoreMesh(core_axis_name="c", subcore_axis_name="s")

    @functools.partial(
        pl.kernel,
        out_type=(jax.ShapeDtypeStruct((e_pad,), F32),
                  jax.ShapeDtypeStruct((NW, LANES), F32)),
        mesh=mesh,
        scratch_types=[
            pltpu.VMEM((NG, 128), jnp.int32),
            pltpu.VMEM((NG, 128), jnp.int32),
            pltpu.VMEM((KC, H), F32),
            pltpu.VMEM((KC, H), F32),
            pltpu.VMEM((KC,), F32),
            pltpu.VMEM((LANES,), F32),
            pltpu.VMEM((LANES,), F32),
            pltpu.SemaphoreType.DMA,
            pltpu.SemaphoreType.DMA,
        ],
        compiler_params=pltpu.CompilerParams(use_tc_tiling_on_sc=False,
                                             needs_layout_passes=False),
    )
    def k(src_h, dst_h, xl_h, xr_h, att_h, alpha_h, wmax_h,
          srcb, dstb, xls, xrd, alph, attv, mxv, sem1, sem2):
        wid = lax.axis_index("c") * 16 + lax.axis_index("s")
        pltpu.sync_copy(att_h, attv)
        attr = attv[...]
        atts = [attr[h] for h in range(H)]
        iota = lax.iota(jnp.int32, LANES)
        cols = [jnp.full((LANES,), h, jnp.int32) for h in range(H)]

        def chunk(t, mx):
            gc = wid * cpw + t
            e0 = gc * KC
            r0 = gc * NG
            pltpu.sync_copy(src_h.at[pl.ds(r0, NG)], srcb)
            pltpu.sync_copy(dst_h.at[pl.ds(r0, NG)], dstb)
            cps = []
            for g in range(NG):
                cps.append(pltpu.async_copy(
                    xl_h.at[srcb.at[g]], xls.at[pl.ds(g * 128, 128)], sem1))
                cps.append(pltpu.async_copy(
                    xr_h.at[dstb.at[g]], xrd.at[pl.ds(g * 128, 128)], sem2))
            for c in cps:
                c.wait()

            def ebody(j, m):
                eb = j * LANES
                ridx = eb + iota
                acc = jnp.zeros((LANES,), F32)
                for h in range(H):
                    vl = plsc.load_gather(xls, [ridx, cols[h]])
                    vr = plsc.load_gather(xrd, [ridx, cols[h]])
                    v = vl + vr
                    v = jnp.where(v > 0.0, v, 0.2 * v)
                    acc = acc + atts[h] * v
                acc = jnp.where(e0 + ridx < e2, acc, NEG)
                alph[pl.ds(eb, LANES)] = acc
                return jnp.maximum(m, acc)

            mx = lax.fori_loop(0, KC // LANES, ebody, mx)
            pltpu.sync_copy(alph, alpha_h.at[pl.ds(e0, KC)])
            return mx

        mx = lax.fori_loop(0, cpw, chunk, jnp.full((LANES,), NEG, F32))
        mxv[...] = mx
        pltpu.sync_copy(mxv, wmax_h.at[wid])

    return k(src2d, dst2d, xl, xr, att)


def _sc_scatter(src2d, dst2d, alpha, gmax, xl, zrows, zden, e2, cpw):
    mesh = plsc.VectorSubcoreMesh(core_axis_name="c", subcore_axis_name="s")

    @functools.partial(
        pl.kernel,
        out_type=(jax.ShapeDtypeStruct((2, N_PAD, H), F32),
                  jax.ShapeDtypeStruct((2, N_PAD), F32)),
        mesh=mesh,
        scratch_types=[
            pltpu.VMEM((NGB, 128), jnp.int32),
            pltpu.VMEM((NGB, 128), jnp.int32),
            pltpu.VMEM((KCB,), F32),
            pltpu.VMEM((KCB,), F32),
            pltpu.VMEM((KCB, H), F32),
            pltpu.VMEM((LANES,), F32),
            pltpu.SemaphoreType.DMA,
            pltpu.VMEM_SHARED((N_PAD, H), F32),
            pltpu.VMEM_SHARED((N_PAD,), F32),
        ],
        compiler_params=pltpu.CompilerParams(use_tc_tiling_on_sc=False,
                                             needs_layout_passes=False),
    )
    def k(src_h, dst_h, alpha_h, gmax_h, xl_h, zr_h, zd_h, outp_h, denp_h,
          srcb, dstb, alph, ab, rows, gmv, sem1, out_sp, den_sp):
        cid = lax.axis_index("c")
        sid = lax.axis_index("s")
        wid = cid * 16 + sid
        sl = pl.ds(sid * RPS, RPS)
        pltpu.sync_copy(zr_h, out_sp.at[sl])
        pltpu.sync_copy(zd_h, den_sp.at[sl])
        pltpu.sync_copy(gmax_h, gmv)
        plsc.subcore_barrier()
        gm = gmv[...]
        iota = lax.iota(jnp.int32, LANES)
        cols = [jnp.full((LANES,), h, jnp.int32) for h in range(H)]

        def chunk(t, c):
            gc = wid * cpw + t
            e0 = gc * KCB
            r0 = gc * NGB
            pltpu.sync_copy(src_h.at[pl.ds(r0, NGB)], srcb)
            pltpu.sync_copy(dst_h.at[pl.ds(r0, NGB)], dstb)
            pltpu.sync_copy(alpha_h.at[pl.ds(e0, KCB)], alph)

            cps = [pltpu.async_copy(
                xl_h.at[srcb.at[g]], rows.at[pl.ds(g * 128, 128)], sem1)
                for g in range(NGB)]
            for cc in cps:
                cc.wait()

            def sbody(j, c2):
                eb = j * LANES
                ridx = eb + iota
                av = jnp.exp(alph[pl.ds(eb, LANES)] - gm)
                ab[pl.ds(eb, LANES)] = av
                for h in range(H):
                    col = plsc.load_gather(rows, [ridx, cols[h]]) * av
                    plsc.store_scatter(rows, [ridx, cols[h]], col)
                return c2

            lax.fori_loop(0, KCB // LANES, sbody, 0)

            for g in range(NGB):
                pltpu.sync_copy(rows.at[pl.ds(g * 128, 128)],
                                out_sp.at[dstb.at[g]], add=True)
                pltpu.sync_copy(ab.at[pl.ds(g * 128, 128)],
                                den_sp.at[dstb.at[g]], add=True)
            return c

        lax.fori_loop(0, cpw, chunk, 0)
        plsc.subcore_barrier()
        pltpu.sync_copy(out_sp.at[sl], outp_h.at[cid, sl])
        pltpu.sync_copy(den_sp.at[sl], denp_h.at[cid, sl])

    return k(src2d, dst2d, alpha, gmax, xl, zrows, zden)


def _tc_stage0(x_pad, par):
    def body(x_ref, p_ref, xl_ref, xr_ref):
        p = p_ref[...]
        xv = jnp.dot(x_ref[...], jnp.ones((1, H), F32),
                     preferred_element_type=F32)
        xl_ref[...] = xv * p[0:1, :] + p[1:2, :]
        xr_ref[...] = xv * p[2:3, :] + p[3:4, :]

    return pl.pallas_call(
        body,
        grid=(NB,),
        in_specs=[pl.BlockSpec((BLK, 1), lambda i: (i, 0)),
                  pl.BlockSpec((8, H), lambda i: (0, 0))],
        out_specs=[pl.BlockSpec((BLK, H), lambda i: (i, 0))] * 2,
        out_shape=(jax.ShapeDtypeStruct((N_PAD, H), F32),) * 2,
    )(x_pad, par)


def _combine(o0, o1, d0, d1, cb):
    den = jnp.dot(d0 + d1 + 1e-16, jnp.ones((1, H), F32),
                  preferred_element_type=F32)
    return jax.nn.relu((o0 + o1) / den + cb)


def _tc_stage1(o0, o1, d0, d1, par):
    def body(o0_ref, o1_ref, d0_ref, d1_ref, p_ref, xl_ref, xr_ref):
        p = p_ref[...]
        h = _combine(o0_ref[...], o1_ref[...], d0_ref[...], d1_ref[...],
                     p[0:1, :])
        h = jnp.dot(h, p[1:17, :], preferred_element_type=F32) + p[17:18, :]
        xl_ref[...] = jnp.dot(h, p[18:34, :],
                              preferred_element_type=F32) + p[34:35, :]
        xr_ref[...] = jnp.dot(h, p[35:51, :],
                              preferred_element_type=F32) + p[51:52, :]

    return pl.pallas_call(
        body,
        grid=(NB,),
        in_specs=[pl.BlockSpec((BLK, H), lambda i: (i, 0)),
                  pl.BlockSpec((BLK, H), lambda i: (i, 0)),
                  pl.BlockSpec((BLK, 1), lambda i: (i, 0)),
                  pl.BlockSpec((BLK, 1), lambda i: (i, 0)),
                  pl.BlockSpec((56, H), lambda i: (0, 0))],
        out_specs=[pl.BlockSpec((BLK, H), lambda i: (i, 0))] * 2,
        out_shape=(jax.ShapeDtypeStruct((N_PAD, H), F32),) * 2,
    )(o0, o1, d0, d1, par)


def _tc_stage2(o0, o1, d0, d1, bf, par):
    def body(o0_ref, o1_ref, d0_ref, d1_ref, bf_ref, p_ref, out_ref, acc_ref):
        i = pl.program_id(0)
        p = p_ref[...]
        h = _combine(o0_ref[...], o1_ref[...], d0_ref[...], d1_ref[...],
                     p[0:1, :])
        h = jnp.dot(h, p[1:17, :], preferred_element_type=F32) + p[17:18, :]
        gi = lax.broadcasted_iota(jnp.int32, (1, G), 1).astype(F32)
        bfb = jnp.dot(bf_ref[...], jnp.ones((1, G), F32),
                      preferred_element_type=F32)
        oh = jnp.where(bfb == gi, 1.0, 0.0)
        contrib = lax.dot_general(oh, h, (((0,), (0,)), ((), ())),
                                  preferred_element_type=F32)

        @pl.when(i == 0)
        def _():
            acc_ref[...] = contrib

        @pl.when(i > 0)
        def _():
            acc_ref[...] += contrib

        @pl.when(i == NB - 1)
        def _():
            gacc = acc_ref[...]
            gacc = jax.nn.relu(jnp.dot(gacc, p[18:34, :],
                                       preferred_element_type=F32)
                               + p[34:35, :])
            gacc = jax.nn.relu(jnp.dot(gacc, p[35:51, :],
                                       preferred_element_type=F32)
                               + p[51:52, :])
            y = jnp.sum(gacc * p[52:53, :], axis=1, keepdims=True)
            out_ref[...] = y + p[53, 0]

    return pl.pallas_call(
        body,
        grid=(NB,),
        in_specs=[pl.BlockSpec((BLK, H), lambda i: (i, 0)),
                  pl.BlockSpec((BLK, H), lambda i: (i, 0)),
                  pl.BlockSpec((BLK, 1), lambda i: (i, 0)),
                  pl.BlockSpec((BLK, 1), lambda i: (i, 0)),
                  pl.BlockSpec((BLK, 1), lambda i: (i, 0)),
                  pl.BlockSpec((56, H), lambda i: (0, 0))],
        out_specs=pl.BlockSpec((G, 1), lambda i: (0, 0)),
        out_shape=jax.ShapeDtypeStruct((G, 1), F32),
        scratch_shapes=[pltpu.VMEM((G, H), F32)],
    )(o0, o1, d0, d1, bf, par)


def _pack(rows, nrows):
    m = jnp.stack([r.astype(F32) for r in rows])
    return jnp.concatenate(
        [m, jnp.zeros((nrows - m.shape[0], H), F32)], axis=0)


def kernel(x, edge_index, batch, params):
    p = params
    ei = edge_index.astype(jnp.int32)
    loop = jnp.arange(N_NODES, dtype=jnp.int32)
    src = jnp.concatenate([ei[0], loop])
    dst = jnp.concatenate([ei[1], loop])
    e2 = src.shape[0]
    cpw = -(-e2 // (NW * KC))
    e_pad = NW * cpw * KC
    src2d = jnp.zeros((e_pad,), jnp.int32).at[:e2].set(src).reshape(-1, 128)
    dst2d = jnp.zeros((e_pad,), jnp.int32).at[:e2].set(dst).reshape(-1, 128)
    x_pad = jnp.zeros((N_PAD, 1), F32).at[:N_NODES].set(x.astype(F32))
    bf = jnp.full((N_PAD, 1), float(G), F32).at[:N_NODES, 0].set(
        batch.astype(F32))
    zrows = jnp.zeros((RPS, H), F32)
    zden = jnp.zeros((RPS,), F32)

    par0 = _pack([p['Wl0'][:, 0], p['bl0'], p['Wr0'][:, 0], p['br0']], 8)
    xl, xr = _tc_stage0(x_pad, par0)

    def edge_layer(xl_t, xr_t, att):
        alpha, wmax = _sc_alpha(src2d, dst2d, xl_t, xr_t,
                                att.astype(F32), e2, cpw)
        gmax = jnp.full((LANES,), jnp.max(wmax), F32)
        outp, denp = _sc_scatter(src2d, dst2d, alpha, gmax, xl_t,
                                 zrows, zden, e2, e_pad // (NW * KCB))
        return (outp[0], outp[1],
                denp[0].reshape(-1, 1), denp[1].reshape(-1, 1))

    o0, o1, d0, d1 = edge_layer(xl, xr, p['att0'])

    par1 = _pack([p['cb0']] + list(p['lin0_W'].T) + [p['lin0_b']]
                 + list(p['Wl1'].T) + [p['bl1']]
                 + list(p['Wr1'].T) + [p['br1']], 56)
    xl1, xr1 = _tc_stage1(o0, o1, d0, d1, par1)

    o0, o1, d0, d1 = edge_layer(xl1, xr1, p['att1'])

    par2 = _pack([p['cb1']] + list(p['lin1_W'].T) + [p['lin1_b']]
                 + list(p['out0_W'].T) + [p['out0_b']]
                 + list(p['out1_W'].T) + [p['out1_b']]
                 + [p['out2_W'][0]]
                 + [jnp.full((H,), p['out2_b'][0], F32)], 56)
    return _tc_stage2(o0, o1, d0, d1, bf, par2)


# speedup vs baseline: 23.1772x; 1.0182x over previous
import functools

import jax
import jax.numpy as jnp
from jax import lax
from jax.experimental import pallas as pl
from jax.experimental.pallas import tpu as pltpu
from jax.experimental.pallas import tpu_sc as plsc

N_NODES = 100000
N_PAD = 100352
H = 16
G = 64
LANES = 16
KC = 2048
NG = KC // 128
KCB = 1024
NGB = KCB // 128
NW = 32
RPS = N_PAD // 16
NEG = -1e30
BLK = 2048
NB = N_PAD // BLK
F32 = jnp.float32


def _sc_alpha(src2d, dst2d, xl, xr, att, e2, cpw):
    e_pad = src2d.shape[0] * 128
    mesh = plsc.VectorSubcoreMesh(core_axis_name="c", subcore_axis_name="s")

    @functools.partial(
        pl.kernel,
        out_type=(jax.ShapeDtypeStruct((e_pad,), F32),
                  jax.ShapeDtypeStruct((NW, LANES), F32)),
        mesh=mesh,
        scratch_types=[
            pltpu.VMEM((NG, 128), jnp.int32),
            pltpu.VMEM((NG, 128), jnp.int32),
            pltpu.VMEM((KC, H), F32),
            pltpu.VMEM((KC, H), F32),
            pltpu.VMEM((KC,), F32),
            pltpu.VMEM((LANES,), F32),
            pltpu.VMEM((LANES,), F32),
            pltpu.SemaphoreType.DMA,
            pltpu.SemaphoreType.DMA,
        ],
        compiler_params=pltpu.CompilerParams(use_tc_tiling_on_sc=False,
                                             needs_layout_passes=False),
    )
    def k(src_h, dst_h, xl_h, xr_h, att_h, alpha_h, wmax_h,
          srcb, dstb, xls, xrd, alph, attv, mxv, sem1, sem2):
        wid = lax.axis_index("c") * 16 + lax.axis_index("s")
        pltpu.sync_copy(att_h, attv)
        attr = attv[...]
        atts = [attr[h] for h in range(H)]
        iota = lax.iota(jnp.int32, LANES)
        cols = [jnp.full((LANES,), h, jnp.int32) for h in range(H)]

        def chunk(t, mx):
            gc = wid * cpw + t
            e0 = gc * KC
            r0 = gc * NG
            pltpu.sync_copy(src_h.at[pl.ds(r0, NG)], srcb)
            pltpu.sync_copy(dst_h.at[pl.ds(r0, NG)], dstb)
            cps = []
            for g in range(NG):
                cps.append(pltpu.async_copy(
                    xl_h.at[srcb.at[g]], xls.at[pl.ds(g * 128, 128)], sem1))
                cps.append(pltpu.async_copy(
                    xr_h.at[dstb.at[g]], xrd.at[pl.ds(g * 128, 128)], sem2))
            for c in cps:
                c.wait()

            def ebody(j, m):
                eb = j * LANES
                ridx = eb + iota
                acc = jnp.zeros((LANES,), F32)
                for h in range(H):
                    vl = plsc.load_gather(xls, [ridx, cols[h]])
                    vr = plsc.load_gather(xrd, [ridx, cols[h]])
                    v = vl + vr
                    v = jnp.where(v > 0.0, v, 0.2 * v)
                    acc = acc + atts[h] * v
                acc = jnp.where(e0 + ridx < e2, acc, NEG)
                alph[pl.ds(eb, LANES)] = acc
                return jnp.maximum(m, acc)

            mx = lax.fori_loop(0, KC // LANES, ebody, mx)
            pltpu.sync_copy(alph, alpha_h.at[pl.ds(e0, KC)])
            return mx

        mx = lax.fori_loop(0, cpw, chunk, jnp.full((LANES,), NEG, F32))
        mxv[...] = mx
        pltpu.sync_copy(mxv, wmax_h.at[wid])

    return k(src2d, dst2d, xl, xr, att)


def _sc_scatter(src2d, dst2d, alpha, gmax, xl, zrows, zden, e2, cpw):
    mesh = plsc.VectorSubcoreMesh(core_axis_name="c", subcore_axis_name="s")

    @functools.partial(
        pl.kernel,
        out_type=(jax.ShapeDtypeStruct((2, N_PAD, H), F32),
                  jax.ShapeDtypeStruct((2, N_PAD), F32)),
        mesh=mesh,
        scratch_types=[
            pltpu.VMEM((NGB, 128), jnp.int32),
            pltpu.VMEM((NGB, 128), jnp.int32),
            pltpu.VMEM((KCB,), F32),
            pltpu.VMEM((KCB,), F32),
            pltpu.VMEM((KCB, H), F32),
            pltpu.VMEM((LANES,), F32),
            pltpu.SemaphoreType.DMA,
            pltpu.SemaphoreType.DMA,
            pltpu.VMEM_SHARED((N_PAD, H), F32),
            pltpu.VMEM_SHARED((N_PAD,), F32),
        ],
        compiler_params=pltpu.CompilerParams(use_tc_tiling_on_sc=False,
                                             needs_layout_passes=False),
    )
    def k(src_h, dst_h, alpha_h, gmax_h, xl_h, zr_h, zd_h, outp_h, denp_h,
          srcb, dstb, alph, ab, rows, gmv, sem1, sem2, out_sp, den_sp):
        cid = lax.axis_index("c")
        sid = lax.axis_index("s")
        wid = cid * 16 + sid
        sl = pl.ds(sid * RPS, RPS)
        pltpu.sync_copy(zr_h, out_sp.at[sl])
        pltpu.sync_copy(zd_h, den_sp.at[sl])
        pltpu.sync_copy(gmax_h, gmv)
        plsc.subcore_barrier()
        gm = gmv[...]
        iota = lax.iota(jnp.int32, LANES)
        cols = [jnp.full((LANES,), h, jnp.int32) for h in range(H)]

        def chunk(t, c):
            gc = wid * cpw + t
            e0 = gc * KCB
            r0 = gc * NGB
            pltpu.sync_copy(src_h.at[pl.ds(r0, NGB)], srcb)
            pltpu.sync_copy(dst_h.at[pl.ds(r0, NGB)], dstb)
            pltpu.sync_copy(alpha_h.at[pl.ds(e0, KCB)], alph)

            cps = [pltpu.async_copy(
                xl_h.at[srcb.at[g]], rows.at[pl.ds(g * 128, 128)], sem1)
                for g in range(NGB)]
            for cc in cps:
                cc.wait()

            def sbody(j, c2):
                eb = j * LANES
                ridx = eb + iota
                av = jnp.exp(alph[pl.ds(eb, LANES)] - gm)
                ab[pl.ds(eb, LANES)] = av
                for h in range(H):
                    col = plsc.load_gather(rows, [ridx, cols[h]]) * av
                    plsc.store_scatter(rows, [ridx, cols[h]], col)
                return c2

            lax.fori_loop(0, KCB // LANES, sbody, 0)

            scs = []
            for g in range(NGB):
                scs.append(pltpu.make_async_copy(
                    rows.at[pl.ds(g * 128, 128)],
                    out_sp.at[dstb.at[g]], sem2))
                scs.append(pltpu.make_async_copy(
                    ab.at[pl.ds(g * 128, 128)],
                    den_sp.at[dstb.at[g]], sem2))
            for d in scs:
                d.start(add=True)
            for d in scs:
                d.wait()
            return c

        lax.fori_loop(0, cpw, chunk, 0)
        plsc.subcore_barrier()
        pltpu.sync_copy(out_sp.at[sl], outp_h.at[cid, sl])
        pltpu.sync_copy(den_sp.at[sl], denp_h.at[cid, sl])

    return k(src2d, dst2d, alpha, gmax, xl, zrows, zden)


def _tc_stage0(x_pad, par):
    def body(x_ref, p_ref, xl_ref, xr_ref):
        p = p_ref[...]
        xv = jnp.dot(x_ref[...], jnp.ones((1, H), F32),
                     preferred_element_type=F32)
        xl_ref[...] = xv * p[0:1, :] + p[1:2, :]
        xr_ref[...] = xv * p[2:3, :] + p[3:4, :]

    return pl.pallas_call(
        body,
        grid=(NB,),
        in_specs=[pl.BlockSpec((BLK, 1), lambda i: (i, 0)),
                  pl.BlockSpec((8, H), lambda i: (0, 0))],
        out_specs=[pl.BlockSpec((BLK, H), lambda i: (i, 0))] * 2,
        out_shape=(jax.ShapeDtypeStruct((N_PAD, H), F32),) * 2,
    )(x_pad, par)


def _combine(o0, o1, d0, d1, cb):
    den = jnp.dot(d0 + d1 + 1e-16, jnp.ones((1, H), F32),
                  preferred_element_type=F32)
    return jax.nn.relu((o0 + o1) / den + cb)


def _tc_stage1(o0, o1, d0, d1, par):
    def body(o0_ref, o1_ref, d0_ref, d1_ref, p_ref, xl_ref, xr_ref):
        p = p_ref[...]
        h = _combine(o0_ref[...], o1_ref[...], d0_ref[...], d1_ref[...],
                     p[0:1, :])
        h = jnp.dot(h, p[1:17, :], preferred_element_type=F32) + p[17:18, :]
        xl_ref[...] = jnp.dot(h, p[18:34, :],
                              preferred_element_type=F32) + p[34:35, :]
        xr_ref[...] = jnp.dot(h, p[35:51, :],
                              preferred_element_type=F32) + p[51:52, :]

    return pl.pallas_call(
        body,
        grid=(NB,),
        in_specs=[pl.BlockSpec((BLK, H), lambda i: (i, 0)),
                  pl.BlockSpec((BLK, H), lambda i: (i, 0)),
                  pl.BlockSpec((BLK, 1), lambda i: (i, 0)),
                  pl.BlockSpec((BLK, 1), lambda i: (i, 0)),
                  pl.BlockSpec((56, H), lambda i: (0, 0))],
        out_specs=[pl.BlockSpec((BLK, H), lambda i: (i, 0))] * 2,
        out_shape=(jax.ShapeDtypeStruct((N_PAD, H), F32),) * 2,
    )(o0, o1, d0, d1, par)


def _tc_stage2(o0, o1, d0, d1, bf, par):
    def body(o0_ref, o1_ref, d0_ref, d1_ref, bf_ref, p_ref, out_ref, acc_ref):
        i = pl.program_id(0)
        p = p_ref[...]
        h = _combine(o0_ref[...], o1_ref[...], d0_ref[...], d1_ref[...],
                     p[0:1, :])
        h = jnp.dot(h, p[1:17, :], preferred_element_type=F32) + p[17:18, :]
        gi = lax.broadcasted_iota(jnp.int32, (1, G), 1).astype(F32)
        bfb = jnp.dot(bf_ref[...], jnp.ones((1, G), F32),
                      preferred_element_type=F32)
        oh = jnp.where(bfb == gi, 1.0, 0.0)
        contrib = lax.dot_general(oh, h, (((0,), (0,)), ((), ())),
                                  preferred_element_type=F32)

        @pl.when(i == 0)
        def _():
            acc_ref[...] = contrib

        @pl.when(i > 0)
        def _():
            acc_ref[...] += contrib

        @pl.when(i == NB - 1)
        def _():
            gacc = acc_ref[...]
            gacc = jax.nn.relu(jnp.dot(gacc, p[18:34, :],
                                       preferred_element_type=F32)
                               + p[34:35, :])
            gacc = jax.nn.relu(jnp.dot(gacc, p[35:51, :],
                                       preferred_element_type=F32)
                               + p[51:52, :])
            y = jnp.sum(gacc * p[52:53, :], axis=1, keepdims=True)
            out_ref[...] = y + p[53, 0]

    return pl.pallas_call(
        body,
        grid=(NB,),
        in_specs=[pl.BlockSpec((BLK, H), lambda i: (i, 0)),
                  pl.BlockSpec((BLK, H), lambda i: (i, 0)),
                  pl.BlockSpec((BLK, 1), lambda i: (i, 0)),
                  pl.BlockSpec((BLK, 1), lambda i: (i, 0)),
                  pl.BlockSpec((BLK, 1), lambda i: (i, 0)),
                  pl.BlockSpec((56, H), lambda i: (0, 0))],
        out_specs=pl.BlockSpec((G, 1), lambda i: (0, 0)),
        out_shape=jax.ShapeDtypeStruct((G, 1), F32),
        scratch_shapes=[pltpu.VMEM((G, H), F32)],
    )(o0, o1, d0, d1, bf, par)


def _pack(rows, nrows):
    m = jnp.stack([r.astype(F32) for r in rows])
    return jnp.concatenate(
        [m, jnp.zeros((nrows - m.shape[0], H), F32)], axis=0)


def kernel(x, edge_index, batch, params):
    p = params
    ei = edge_index.astype(jnp.int32)
    loop = jnp.arange(N_NODES, dtype=jnp.int32)
    src = jnp.concatenate([ei[0], loop])
    dst = jnp.concatenate([ei[1], loop])
    e2 = src.shape[0]
    cpw = -(-e2 // (NW * KC))
    e_pad = NW * cpw * KC
    src2d = jnp.zeros((e_pad,), jnp.int32).at[:e2].set(src).reshape(-1, 128)
    dst2d = jnp.zeros((e_pad,), jnp.int32).at[:e2].set(dst).reshape(-1, 128)
    x_pad = jnp.zeros((N_PAD, 1), F32).at[:N_NODES].set(x.astype(F32))
    bf = jnp.full((N_PAD, 1), float(G), F32).at[:N_NODES, 0].set(
        batch.astype(F32))
    zrows = jnp.zeros((RPS, H), F32)
    zden = jnp.zeros((RPS,), F32)

    par0 = _pack([p['Wl0'][:, 0], p['bl0'], p['Wr0'][:, 0], p['br0']], 8)
    xl, xr = _tc_stage0(x_pad, par0)

    def edge_layer(xl_t, xr_t, att):
        alpha, wmax = _sc_alpha(src2d, dst2d, xl_t, xr_t,
                                att.astype(F32), e2, cpw)
        gmax = jnp.full((LANES,), jnp.max(wmax), F32)
        outp, denp = _sc_scatter(src2d, dst2d, alpha, gmax, xl_t,
                                 zrows, zden, e2, e_pad // (NW * KCB))
        return (outp[0], outp[1],
                denp[0].reshape(-1, 1), denp[1].reshape(-1, 1))

    o0, o1, d0, d1 = edge_layer(xl, xr, p['att0'])

    par1 = _pack([p['cb0']] + list(p['lin0_W'].T) + [p['lin0_b']]
                 + list(p['Wl1'].T) + [p['bl1']]
                 + list(p['Wr1'].T) + [p['br1']], 56)
    xl1, xr1 = _tc_stage1(o0, o1, d0, d1, par1)

    o0, o1, d0, d1 = edge_layer(xl1, xr1, p['att1'])

    par2 = _pack([p['cb1']] + list(p['lin1_W'].T) + [p['lin1_b']]
                 + list(p['out0_W'].T) + [p['out0_b']]
                 + list(p['out1_W'].T) + [p['out1_b']]
                 + [p['out2_W'][0]]
                 + [jnp.full((H,), p['out2_b'][0], F32)], 56)
    return _tc_stage2(o0, o1, d0, d1, bf, par2)


# speedup vs baseline: 23.2330x; 1.0024x over previous
import functools

import jax
import jax.numpy as jnp
from jax import lax
from jax.experimental import pallas as pl
from jax.experimental.pallas import tpu as pltpu
from jax.experimental.pallas import tpu_sc as plsc

N_NODES = 100000
N_PAD = 100352
H = 16
G = 64
LANES = 16
KC = 3072
NG = KC // 128
KCB = 1024
NGB = KCB // 128
NW = 32
RPS = N_PAD // 16
NEG = -1e30
BLK = 2048
NB = N_PAD // BLK
F32 = jnp.float32


def _sc_alpha(src2d, dst2d, xl, xr, att, e2, cpw):
    e_pad = src2d.shape[0] * 128
    mesh = plsc.VectorSubcoreMesh(core_axis_name="c", subcore_axis_name="s")

    @functools.partial(
        pl.kernel,
        out_type=(jax.ShapeDtypeStruct((e_pad,), F32),
                  jax.ShapeDtypeStruct((NW, LANES), F32)),
        mesh=mesh,
        scratch_types=[
            pltpu.VMEM((NG, 128), jnp.int32),
            pltpu.VMEM((NG, 128), jnp.int32),
            pltpu.VMEM((KC, H), F32),
            pltpu.VMEM((KC, H), F32),
            pltpu.VMEM((KC,), F32),
            pltpu.VMEM((LANES,), F32),
            pltpu.VMEM((LANES,), F32),
            pltpu.SemaphoreType.DMA,
            pltpu.SemaphoreType.DMA,
        ],
        compiler_params=pltpu.CompilerParams(use_tc_tiling_on_sc=False,
                                             needs_layout_passes=False),
    )
    def k(src_h, dst_h, xl_h, xr_h, att_h, alpha_h, wmax_h,
          srcb, dstb, xls, xrd, alph, attv, mxv, sem1, sem2):
        wid = lax.axis_index("c") * 16 + lax.axis_index("s")
        pltpu.sync_copy(att_h, attv)
        attr = attv[...]
        atts = [attr[h] for h in range(H)]
        iota = lax.iota(jnp.int32, LANES)
        cols = [jnp.full((LANES,), h, jnp.int32) for h in range(H)]

        def chunk(t, mx):
            gc = wid * cpw + t
            e0 = gc * KC
            r0 = gc * NG
            pltpu.sync_copy(src_h.at[pl.ds(r0, NG)], srcb)
            pltpu.sync_copy(dst_h.at[pl.ds(r0, NG)], dstb)
            cps = []
            for g in range(NG):
                cps.append(pltpu.async_copy(
                    xl_h.at[srcb.at[g]], xls.at[pl.ds(g * 128, 128)], sem1))
                cps.append(pltpu.async_copy(
                    xr_h.at[dstb.at[g]], xrd.at[pl.ds(g * 128, 128)], sem2))
            for cc in cps:
                cc.wait()

            def ebody(jb, m2):
                eb = jb * LANES
                ridx = eb + iota
                acc = jnp.zeros((LANES,), F32)
                for h in range(H):
                    vl = plsc.load_gather(xls, [ridx, cols[h]])
                    vr = plsc.load_gather(xrd, [ridx, cols[h]])
                    v = vl + vr
                    v = jnp.where(v > 0.0, v, 0.2 * v)
                    acc = acc + atts[h] * v
                acc = jnp.where(e0 + ridx < e2, acc, NEG)
                alph[pl.ds(eb, LANES)] = acc
                return jnp.maximum(m2, acc)

            mx = lax.fori_loop(0, KC // LANES, ebody, mx)
            pltpu.sync_copy(alph, alpha_h.at[pl.ds(e0, KC)])
            return mx

        mx = lax.fori_loop(0, cpw, chunk, jnp.full((LANES,), NEG, F32))
        mxv[...] = mx
        pltpu.sync_copy(mxv, wmax_h.at[wid])

    return k(src2d, dst2d, xl, xr, att)


def _sc_scatter(src2d, dst2d, alpha, gmax, xl, zrows, zden, e2, cpw):
    mesh = plsc.VectorSubcoreMesh(core_axis_name="c", subcore_axis_name="s")

    @functools.partial(
        pl.kernel,
        out_type=(jax.ShapeDtypeStruct((2, N_PAD, H), F32),
                  jax.ShapeDtypeStruct((2, N_PAD), F32)),
        mesh=mesh,
        scratch_types=[
            pltpu.VMEM((NGB, 128), jnp.int32),
            pltpu.VMEM((NGB, 128), jnp.int32),
            pltpu.VMEM((KCB,), F32),
            pltpu.VMEM((KCB,), F32),
            pltpu.VMEM((KCB, H), F32),
            pltpu.VMEM((LANES,), F32),
            pltpu.SemaphoreType.DMA,
            pltpu.SemaphoreType.DMA,
            pltpu.VMEM_SHARED((N_PAD, H), F32),
            pltpu.VMEM_SHARED((N_PAD,), F32),
        ],
        compiler_params=pltpu.CompilerParams(use_tc_tiling_on_sc=False,
                                             needs_layout_passes=False),
    )
    def k(src_h, dst_h, alpha_h, gmax_h, xl_h, zr_h, zd_h, outp_h, denp_h,
          srcb, dstb, alph, ab, rows, gmv, sem1, sem2, out_sp, den_sp):
        cid = lax.axis_index("c")
        sid = lax.axis_index("s")
        wid = cid * 16 + sid
        sl = pl.ds(sid * RPS, RPS)
        pltpu.sync_copy(zr_h, out_sp.at[sl])
        pltpu.sync_copy(zd_h, den_sp.at[sl])
        pltpu.sync_copy(gmax_h, gmv)
        plsc.subcore_barrier()
        gm = gmv[...]
        iota = lax.iota(jnp.int32, LANES)
        cols = [jnp.full((LANES,), h, jnp.int32) for h in range(H)]

        def chunk(t, c):
            gc = wid * cpw + t
            e0 = gc * KCB
            r0 = gc * NGB
            pltpu.sync_copy(src_h.at[pl.ds(r0, NGB)], srcb)
            pltpu.sync_copy(dst_h.at[pl.ds(r0, NGB)], dstb)
            pltpu.sync_copy(alpha_h.at[pl.ds(e0, KCB)], alph)

            cps = [pltpu.async_copy(
                xl_h.at[srcb.at[g]], rows.at[pl.ds(g * 128, 128)], sem1)
                for g in range(NGB)]
            for cc in cps:
                cc.wait()

            def sbody(jb, c3):
                eb = jb * LANES
                ridx = eb + iota
                av = jnp.exp(alph[pl.ds(eb, LANES)] - gm)
                ab[pl.ds(eb, LANES)] = av
                for h in range(H):
                    col = plsc.load_gather(rows, [ridx, cols[h]]) * av
                    plsc.store_scatter(rows, [ridx, cols[h]], col)
                return c3

            lax.fori_loop(0, KCB // LANES, sbody, 0)

            scs = []
            for g in range(NGB):
                scs.append(pltpu.make_async_copy(
                    rows.at[pl.ds(g * 128, 128)],
                    out_sp.at[dstb.at[g]], sem2))
                scs.append(pltpu.make_async_copy(
                    ab.at[pl.ds(g * 128, 128)],
                    den_sp.at[dstb.at[g]], sem2))
            for d in scs:
                d.start(add=True)
            for d in scs:
                d.wait()
            return c

        lax.fori_loop(0, cpw, chunk, 0)
        plsc.subcore_barrier()
        pltpu.sync_copy(out_sp.at[sl], outp_h.at[cid, sl])
        pltpu.sync_copy(den_sp.at[sl], denp_h.at[cid, sl])

    return k(src2d, dst2d, alpha, gmax, xl, zrows, zden)


def _tc_stage0(x_pad, par):
    def body(x_ref, p_ref, xl_ref, xr_ref):
        p = p_ref[...]
        xv = jnp.dot(x_ref[...], jnp.ones((1, H), F32),
                     preferred_element_type=F32)
        xl_ref[...] = xv * p[0:1, :] + p[1:2, :]
        xr_ref[...] = xv * p[2:3, :] + p[3:4, :]

    return pl.pallas_call(
        body,
        grid=(NB,),
        in_specs=[pl.BlockSpec((BLK, 1), lambda i: (i, 0)),
                  pl.BlockSpec((8, H), lambda i: (0, 0))],
        out_specs=[pl.BlockSpec((BLK, H), lambda i: (i, 0))] * 2,
        out_shape=(jax.ShapeDtypeStruct((N_PAD, H), F32),) * 2,
    )(x_pad, par)


def _combine(o0, o1, d0, d1, cb):
    den = jnp.dot(d0 + d1 + 1e-16, jnp.ones((1, H), F32),
                  preferred_element_type=F32)
    return jax.nn.relu((o0 + o1) / den + cb)


def _tc_stage1(o0, o1, d0, d1, par):
    def body(o0_ref, o1_ref, d0_ref, d1_ref, p_ref, xl_ref, xr_ref):
        p = p_ref[...]
        h = _combine(o0_ref[...], o1_ref[...], d0_ref[...], d1_ref[...],
                     p[0:1, :])
        h = jnp.dot(h, p[1:17, :], preferred_element_type=F32) + p[17:18, :]
        xl_ref[...] = jnp.dot(h, p[18:34, :],
                              preferred_element_type=F32) + p[34:35, :]
        xr_ref[...] = jnp.dot(h, p[35:51, :],
                              preferred_element_type=F32) + p[51:52, :]

    return pl.pallas_call(
        body,
        grid=(NB,),
        in_specs=[pl.BlockSpec((BLK, H), lambda i: (i, 0)),
                  pl.BlockSpec((BLK, H), lambda i: (i, 0)),
                  pl.BlockSpec((BLK, 1), lambda i: (i, 0)),
                  pl.BlockSpec((BLK, 1), lambda i: (i, 0)),
                  pl.BlockSpec((56, H), lambda i: (0, 0))],
        out_specs=[pl.BlockSpec((BLK, H), lambda i: (i, 0))] * 2,
        out_shape=(jax.ShapeDtypeStruct((N_PAD, H), F32),) * 2,
    )(o0, o1, d0, d1, par)


def _tc_stage2(o0, o1, d0, d1, bf, par):
    def body(o0_ref, o1_ref, d0_ref, d1_ref, bf_ref, p_ref, out_ref, acc_ref):
        i = pl.program_id(0)
        p = p_ref[...]
        h = _combine(o0_ref[...], o1_ref[...], d0_ref[...], d1_ref[...],
                     p[0:1, :])
        h = jnp.dot(h, p[1:17, :], preferred_element_type=F32) + p[17:18, :]
        gi = lax.broadcasted_iota(jnp.int32, (1, G), 1).astype(F32)
        bfb = jnp.dot(bf_ref[...], jnp.ones((1, G), F32),
                      preferred_element_type=F32)
        oh = jnp.where(bfb == gi, 1.0, 0.0)
        contrib = lax.dot_general(oh, h, (((0,), (0,)), ((), ())),
                                  preferred_element_type=F32)

        @pl.when(i == 0)
        def _():
            acc_ref[...] = contrib

        @pl.when(i > 0)
        def _():
            acc_ref[...] += contrib

        @pl.when(i == NB - 1)
        def _():
            gacc = acc_ref[...]
            gacc = jax.nn.relu(jnp.dot(gacc, p[18:34, :],
                                       preferred_element_type=F32)
                               + p[34:35, :])
            gacc = jax.nn.relu(jnp.dot(gacc, p[35:51, :],
                                       preferred_element_type=F32)
                               + p[51:52, :])
            y = jnp.sum(gacc * p[52:53, :], axis=1, keepdims=True)
            out_ref[...] = y + p[53, 0]

    return pl.pallas_call(
        body,
        grid=(NB,),
        in_specs=[pl.BlockSpec((BLK, H), lambda i: (i, 0)),
                  pl.BlockSpec((BLK, H), lambda i: (i, 0)),
                  pl.BlockSpec((BLK, 1), lambda i: (i, 0)),
                  pl.BlockSpec((BLK, 1), lambda i: (i, 0)),
                  pl.BlockSpec((BLK, 1), lambda i: (i, 0)),
                  pl.BlockSpec((56, H), lambda i: (0, 0))],
        out_specs=pl.BlockSpec((G, 1), lambda i: (0, 0)),
        out_shape=jax.ShapeDtypeStruct((G, 1), F32),
        scratch_shapes=[pltpu.VMEM((G, H), F32)],
    )(o0, o1, d0, d1, bf, par)


def _pack(rows, nrows):
    m = jnp.stack([r.astype(F32) for r in rows])
    return jnp.concatenate(
        [m, jnp.zeros((nrows - m.shape[0], H), F32)], axis=0)


def kernel(x, edge_index, batch, params):
    p = params
    ei = edge_index.astype(jnp.int32)
    loop = jnp.arange(N_NODES, dtype=jnp.int32)
    src = jnp.concatenate([ei[0], loop])
    dst = jnp.concatenate([ei[1], loop])
    e2 = src.shape[0]
    cpw = -(-e2 // (NW * KC))
    e_pad = NW * cpw * KC
    src2d = jnp.zeros((e_pad,), jnp.int32).at[:e2].set(src).reshape(-1, 128)
    dst2d = jnp.zeros((e_pad,), jnp.int32).at[:e2].set(dst).reshape(-1, 128)
    x_pad = jnp.zeros((N_PAD, 1), F32).at[:N_NODES].set(x.astype(F32))
    bf = jnp.full((N_PAD, 1), float(G), F32).at[:N_NODES, 0].set(
        batch.astype(F32))
    zrows = jnp.zeros((RPS, H), F32)
    zden = jnp.zeros((RPS,), F32)

    par0 = _pack([p['Wl0'][:, 0], p['bl0'], p['Wr0'][:, 0], p['br0']], 8)
    xl, xr = _tc_stage0(x_pad, par0)

    def edge_layer(xl_t, xr_t, att):
        alpha, wmax = _sc_alpha(src2d, dst2d, xl_t, xr_t,
                                att.astype(F32), e2, cpw)
        gmax = jnp.full((LANES,), jnp.max(wmax), F32)
        outp, denp = _sc_scatter(src2d, dst2d, alpha, gmax, xl_t,
                                 zrows, zden, e2, e_pad // (NW * KCB))
        return (outp[0], outp[1],
                denp[0].reshape(-1, 1), denp[1].reshape(-1, 1))

    o0, o1, d0, d1 = edge_layer(xl, xr, p['att0'])

    par1 = _pack([p['cb0']] + list(p['lin0_W'].T) + [p['lin0_b']]
                 + list(p['Wl1'].T) + [p['bl1']]
                 + list(p['Wr1'].T) + [p['br1']], 56)
    xl1, xr1 = _tc_stage1(o0, o1, d0, d1, par1)

    o0, o1, d0, d1 = edge_layer(xl1, xr1, p['att1'])

    par2 = _pack([p['cb1']] + list(p['lin1_W'].T) + [p['lin1_b']]
                 + list(p['out0_W'].T) + [p['out0_b']]
                 + list(p['out1_W'].T) + [p['out1_b']]
                 + [p['out2_W'][0]]
                 + [jnp.full((H,), p['out2_b'][0], F32)], 56)
    return _tc_stage2(o0, o1, d0, d1, bf, par2)


# speedup vs baseline: 26.5997x; 1.1449x over previous
import functools

import jax
import jax.numpy as jnp
from jax import lax
from jax.experimental import pallas as pl
from jax.experimental.pallas import tpu as pltpu
from jax.experimental.pallas import tpu_sc as plsc

N_NODES = 100000
N_PAD = 100352
H = 16
G = 64
LANES = 16
KC = 3072
NG = KC // 128
KCB = 1024
NGB = KCB // 128
NW = 32
RPS = N_PAD // 16
NEG = -1e30
BLK = 2048
NB = N_PAD // BLK
F32 = jnp.float32


def _sc_alpha(src2d, dst2d, xl, xr, att, e2, cpw):
    e_pad = src2d.shape[0] * 128
    mesh = plsc.VectorSubcoreMesh(core_axis_name="c", subcore_axis_name="s")

    @functools.partial(
        pl.kernel,
        out_type=(jax.ShapeDtypeStruct((e_pad,), F32),
                  jax.ShapeDtypeStruct((NW, LANES), F32)),
        mesh=mesh,
        scratch_types=[
            pltpu.VMEM((NG, 128), jnp.int32),
            pltpu.VMEM((NG, 128), jnp.int32),
            pltpu.VMEM((KC, H), F32),
            pltpu.VMEM((KC, H), F32),
            pltpu.VMEM((KC,), F32),
            pltpu.VMEM((LANES,), F32),
            pltpu.VMEM((LANES,), F32),
            pltpu.SemaphoreType.DMA,
            pltpu.SemaphoreType.DMA,
        ],
        compiler_params=pltpu.CompilerParams(use_tc_tiling_on_sc=False,
                                             needs_layout_passes=False),
    )
    def k(src_h, dst_h, xl_h, xr_h, att_h, alpha_h, wmax_h,
          srcb, dstb, xls, xrd, alph, attv, mxv, sem1, sem2):
        wid = lax.axis_index("c") * 16 + lax.axis_index("s")
        pltpu.sync_copy(att_h, attv)
        attr = attv[...]
        atts = [attr[h] for h in range(H)]
        iota = lax.iota(jnp.int32, LANES)
        cols = [jnp.full((LANES,), h, jnp.int32) for h in range(H)]

        def chunk(t, mx):
            gc = wid * cpw + t
            e0 = gc * KC
            r0 = gc * NG
            pltpu.sync_copy(src_h.at[pl.ds(r0, NG)], srcb)
            pltpu.sync_copy(dst_h.at[pl.ds(r0, NG)], dstb)
            cps = []
            for g in range(NG):
                cps.append(pltpu.async_copy(
                    xl_h.at[srcb.at[g]], xls.at[pl.ds(g * 128, 128)], sem1))
                cps.append(pltpu.async_copy(
                    xr_h.at[dstb.at[g]], xrd.at[pl.ds(g * 128, 128)], sem2))

            def ebody(jb, m2):
                eb = jb * LANES
                ridx = eb + iota
                acc = jnp.zeros((LANES,), F32)
                for h in range(H):
                    vl = plsc.load_gather(xls, [ridx, cols[h]])
                    vr = plsc.load_gather(xrd, [ridx, cols[h]])
                    v = vl + vr
                    v = jnp.where(v > 0.0, v, 0.2 * v)
                    acc = acc + atts[h] * v
                acc = jnp.where(e0 + ridx < e2, acc, NEG)
                alph[pl.ds(eb, LANES)] = acc
                return jnp.maximum(m2, acc)

            gpj = 128 // LANES
            for g in range(NG):
                cps[2 * g].wait()
                cps[2 * g + 1].wait()
                mx = lax.fori_loop(g * gpj, (g + 1) * gpj, ebody, mx)
            pltpu.sync_copy(alph, alpha_h.at[pl.ds(e0, KC)])
            return mx

        mx = lax.fori_loop(0, cpw, chunk, jnp.full((LANES,), NEG, F32))
        mxv[...] = mx
        pltpu.sync_copy(mxv, wmax_h.at[wid])

    return k(src2d, dst2d, xl, xr, att)


def _sc_scatter(src2d, dst2d, alpha, gmax, xl, zrows, zden, e2, cpw):
    mesh = plsc.VectorSubcoreMesh(core_axis_name="c", subcore_axis_name="s")

    @functools.partial(
        pl.kernel,
        out_type=(jax.ShapeDtypeStruct((2, N_PAD, H), F32),
                  jax.ShapeDtypeStruct((2, N_PAD), F32)),
        mesh=mesh,
        scratch_types=[
            pltpu.VMEM((NGB, 128), jnp.int32),
            pltpu.VMEM((NGB, 128), jnp.int32),
            pltpu.VMEM((KCB,), F32),
            pltpu.VMEM((KCB,), F32),
            pltpu.VMEM((KCB, H), F32),
            pltpu.VMEM((LANES,), F32),
            pltpu.SemaphoreType.DMA,
            pltpu.SemaphoreType.DMA,
            pltpu.VMEM_SHARED((N_PAD, H), F32),
            pltpu.VMEM_SHARED((N_PAD,), F32),
        ],
        compiler_params=pltpu.CompilerParams(use_tc_tiling_on_sc=False,
                                             needs_layout_passes=False),
    )
    def k(src_h, dst_h, alpha_h, gmax_h, xl_h, zr_h, zd_h, outp_h, denp_h,
          srcb, dstb, alph, ab, rows, gmv, sem1, sem2, out_sp, den_sp):
        cid = lax.axis_index("c")
        sid = lax.axis_index("s")
        wid = cid * 16 + sid
        sl = pl.ds(sid * RPS, RPS)
        pltpu.sync_copy(zr_h, out_sp.at[sl])
        pltpu.sync_copy(zd_h, den_sp.at[sl])
        pltpu.sync_copy(gmax_h, gmv)
        plsc.subcore_barrier()
        gm = gmv[...]
        iota = lax.iota(jnp.int32, LANES)
        cols = [jnp.full((LANES,), h, jnp.int32) for h in range(H)]

        def chunk(t, c):
            gc = wid * cpw + t
            e0 = gc * KCB
            r0 = gc * NGB
            pltpu.sync_copy(src_h.at[pl.ds(r0, NGB)], srcb)
            pltpu.sync_copy(dst_h.at[pl.ds(r0, NGB)], dstb)
            pltpu.sync_copy(alpha_h.at[pl.ds(e0, KCB)], alph)

            cps = [pltpu.async_copy(
                xl_h.at[srcb.at[g]], rows.at[pl.ds(g * 128, 128)], sem1)
                for g in range(NGB)]

            def sbody(jb, c3):
                eb = jb * LANES
                ridx = eb + iota
                av = jnp.exp(alph[pl.ds(eb, LANES)] - gm)
                ab[pl.ds(eb, LANES)] = av
                for h in range(H):
                    col = plsc.load_gather(rows, [ridx, cols[h]]) * av
                    plsc.store_scatter(rows, [ridx, cols[h]], col)
                return c3

            gpj = 128 // LANES
            for g in range(NGB):
                cps[g].wait()
                lax.fori_loop(g * gpj, (g + 1) * gpj, sbody, 0)

            scs = []
            for g in range(NGB):
                scs.append(pltpu.make_async_copy(
                    rows.at[pl.ds(g * 128, 128)],
                    out_sp.at[dstb.at[g]], sem2))
                scs.append(pltpu.make_async_copy(
                    ab.at[pl.ds(g * 128, 128)],
                    den_sp.at[dstb.at[g]], sem2))
            for d in scs:
                d.start(add=True)
            for d in scs:
                d.wait()
            return c

        lax.fori_loop(0, cpw, chunk, 0)
        plsc.subcore_barrier()
        pltpu.sync_copy(out_sp.at[sl], outp_h.at[cid, sl])
        pltpu.sync_copy(den_sp.at[sl], denp_h.at[cid, sl])

    return k(src2d, dst2d, alpha, gmax, xl, zrows, zden)


def _tc_stage0(x_pad, par):
    def body(x_ref, p_ref, xl_ref, xr_ref):
        p = p_ref[...]
        xv = jnp.dot(x_ref[...], jnp.ones((1, H), F32),
                     preferred_element_type=F32)
        xl_ref[...] = xv * p[0:1, :] + p[1:2, :]
        xr_ref[...] = xv * p[2:3, :] + p[3:4, :]

    return pl.pallas_call(
        body,
        grid=(NB,),
        in_specs=[pl.BlockSpec((BLK, 1), lambda i: (i, 0)),
                  pl.BlockSpec((8, H), lambda i: (0, 0))],
        out_specs=[pl.BlockSpec((BLK, H), lambda i: (i, 0))] * 2,
        out_shape=(jax.ShapeDtypeStruct((N_PAD, H), F32),) * 2,
    )(x_pad, par)


def _combine(o0, o1, d0, d1, cb):
    den = jnp.dot(d0 + d1 + 1e-16, jnp.ones((1, H), F32),
                  preferred_element_type=F32)
    return jax.nn.relu((o0 + o1) / den + cb)


def _tc_stage1(o0, o1, d0, d1, par):
    def body(o0_ref, o1_ref, d0_ref, d1_ref, p_ref, xl_ref, xr_ref):
        p = p_ref[...]
        h = _combine(o0_ref[...], o1_ref[...], d0_ref[...], d1_ref[...],
                     p[0:1, :])
        h = jnp.dot(h, p[1:17, :], preferred_element_type=F32) + p[17:18, :]
        xl_ref[...] = jnp.dot(h, p[18:34, :],
                              preferred_element_type=F32) + p[34:35, :]
        xr_ref[...] = jnp.dot(h, p[35:51, :],
                              preferred_element_type=F32) + p[51:52, :]

    return pl.pallas_call(
        body,
        grid=(NB,),
        in_specs=[pl.BlockSpec((BLK, H), lambda i: (i, 0)),
                  pl.BlockSpec((BLK, H), lambda i: (i, 0)),
                  pl.BlockSpec((BLK, 1), lambda i: (i, 0)),
                  pl.BlockSpec((BLK, 1), lambda i: (i, 0)),
                  pl.BlockSpec((56, H), lambda i: (0, 0))],
        out_specs=[pl.BlockSpec((BLK, H), lambda i: (i, 0))] * 2,
        out_shape=(jax.ShapeDtypeStruct((N_PAD, H), F32),) * 2,
    )(o0, o1, d0, d1, par)


def _tc_stage2(o0, o1, d0, d1, bf, par):
    def body(o0_ref, o1_ref, d0_ref, d1_ref, bf_ref, p_ref, out_ref, acc_ref):
        i = pl.program_id(0)
        p = p_ref[...]
        h = _combine(o0_ref[...], o1_ref[...], d0_ref[...], d1_ref[...],
                     p[0:1, :])
        h = jnp.dot(h, p[1:17, :], preferred_element_type=F32) + p[17:18, :]
        gi = lax.broadcasted_iota(jnp.int32, (1, G), 1).astype(F32)
        bfb = jnp.dot(bf_ref[...], jnp.ones((1, G), F32),
                      preferred_element_type=F32)
        oh = jnp.where(bfb == gi, 1.0, 0.0)
        contrib = lax.dot_general(oh, h, (((0,), (0,)), ((), ())),
                                  preferred_element_type=F32)

        @pl.when(i == 0)
        def _():
            acc_ref[...] = contrib

        @pl.when(i > 0)
        def _():
            acc_ref[...] += contrib

        @pl.when(i == NB - 1)
        def _():
            gacc = acc_ref[...]
            gacc = jax.nn.relu(jnp.dot(gacc, p[18:34, :],
                                       preferred_element_type=F32)
                               + p[34:35, :])
            gacc = jax.nn.relu(jnp.dot(gacc, p[35:51, :],
                                       preferred_element_type=F32)
                               + p[51:52, :])
            y = jnp.sum(gacc * p[52:53, :], axis=1, keepdims=True)
            out_ref[...] = y + p[53, 0]

    return pl.pallas_call(
        body,
        grid=(NB,),
        in_specs=[pl.BlockSpec((BLK, H), lambda i: (i, 0)),
                  pl.BlockSpec((BLK, H), lambda i: (i, 0)),
                  pl.BlockSpec((BLK, 1), lambda i: (i, 0)),
                  pl.BlockSpec((BLK, 1), lambda i: (i, 0)),
                  pl.BlockSpec((BLK, 1), lambda i: (i, 0)),
                  pl.BlockSpec((56, H), lambda i: (0, 0))],
        out_specs=pl.BlockSpec((G, 1), lambda i: (0, 0)),
        out_shape=jax.ShapeDtypeStruct((G, 1), F32),
        scratch_shapes=[pltpu.VMEM((G, H), F32)],
    )(o0, o1, d0, d1, bf, par)


def _pack(rows, nrows):
    m = jnp.stack([r.astype(F32) for r in rows])
    return jnp.concatenate(
        [m, jnp.zeros((nrows - m.shape[0], H), F32)], axis=0)


def kernel(x, edge_index, batch, params):
    p = params
    ei = edge_index.astype(jnp.int32)
    loop = jnp.arange(N_NODES, dtype=jnp.int32)
    src = jnp.concatenate([ei[0], loop])
    dst = jnp.concatenate([ei[1], loop])
    e2 = src.shape[0]
    cpw = -(-e2 // (NW * KC))
    e_pad = NW * cpw * KC
    src2d = jnp.zeros((e_pad,), jnp.int32).at[:e2].set(src).reshape(-1, 128)
    dst2d = jnp.zeros((e_pad,), jnp.int32).at[:e2].set(dst).reshape(-1, 128)
    x_pad = jnp.zeros((N_PAD, 1), F32).at[:N_NODES].set(x.astype(F32))
    bf = jnp.full((N_PAD, 1), float(G), F32).at[:N_NODES, 0].set(
        batch.astype(F32))
    zrows = jnp.zeros((RPS, H), F32)
    zden = jnp.zeros((RPS,), F32)

    par0 = _pack([p['Wl0'][:, 0], p['bl0'], p['Wr0'][:, 0], p['br0']], 8)
    xl, xr = _tc_stage0(x_pad, par0)

    def edge_layer(xl_t, xr_t, att):
        alpha, wmax = _sc_alpha(src2d, dst2d, xl_t, xr_t,
                                att.astype(F32), e2, cpw)
        gmax = jnp.full((LANES,), jnp.max(wmax), F32)
        outp, denp = _sc_scatter(src2d, dst2d, alpha, gmax, xl_t,
                                 zrows, zden, e2, e_pad // (NW * KCB))
        return (outp[0], outp[1],
                denp[0].reshape(-1, 1), denp[1].reshape(-1, 1))

    o0, o1, d0, d1 = edge_layer(xl, xr, p['att0'])

    par1 = _pack([p['cb0']] + list(p['lin0_W'].T) + [p['lin0_b']]
                 + list(p['Wl1'].T) + [p['bl1']]
                 + list(p['Wr1'].T) + [p['br1']], 56)
    xl1, xr1 = _tc_stage1(o0, o1, d0, d1, par1)

    o0, o1, d0, d1 = edge_layer(xl1, xr1, p['att1'])

    par2 = _pack([p['cb1']] + list(p['lin1_W'].T) + [p['lin1_b']]
                 + list(p['out0_W'].T) + [p['out0_b']]
                 + list(p['out1_W'].T) + [p['out1_b']]
                 + [p['out2_W'][0]]
                 + [jnp.full((H,), p['out2_b'][0], F32)], 56)
    return _tc_stage2(o0, o1, d0, d1, bf, par2)
